# baseline (speedup 1.0000x reference)
import jax
import jax.numpy as jnp
from jax.experimental import pallas as pl

_N = 4096
_D = 64
_R = 256
_T = 0.07
_NUM_NEG = _N - 1
_K_TOP = max(1, int(_NUM_NEG * 0.5))
_K_BOT = max(0, int(_NUM_NEG * 0.1))
_N_KEPT = _K_TOP - _K_BOT
_FILL = -10.0
_INT_MIN = -2147483648
_BRK = 256


def _pcnt(mask_a, mask_b):
    c = (jnp.where(mask_a, jnp.int16(1), jnp.int16(0))
         + jnp.where(mask_b, jnp.int16(256), jnp.int16(0)))
    n = c.shape[1]
    for _ in range(4):
        n //= 2
        c = c[:, :n] + c[:, n:]
    c32 = c.astype(jnp.int32)
    ca = jnp.sum(c32 & 255, axis=1, keepdims=True)
    cb = jnp.sum(jax.lax.shift_right_logical(c32, 8), axis=1, keepdims=True)
    return ca, cb


def _loss_kernel(q_ref, k_ref, out_ref):
    i = pl.program_id(0)
    q = q_ref[...]
    k = k_ref[...]
    sim = jax.lax.dot_general(
        q, k, (((1,), (1,)), ((), ())),
        preferred_element_type=jnp.float32,
        precision=jax.lax.Precision.HIGHEST)
    R, N = sim.shape
    row = jax.lax.broadcasted_iota(jnp.int32, (R, N), 0)
    col = jax.lax.broadcasted_iota(jnp.int32, (R, N), 1)
    diag = col == row + i * R
    l_pos = jnp.sum(jnp.where(diag, sim, 0.0), axis=1, keepdims=True)

    bits = jax.lax.bitcast_convert_type(sim, jnp.int32)
    key = jnp.where(bits < 0, bits ^ jnp.int32(0x7FFFFFFF), bits)
    key = jnp.where(diag, jnp.int32(_INT_MIN), key)

    key_hi = jax.lax.shift_right_arithmetic(key, 16).astype(jnp.int16)
    key_lo = (jnp.bitwise_and(key, 0xFFFF) - 32768).astype(jnp.int16)

    need_a = _K_BOT + 1
    need_b = _K_TOP

    ca0, cb0 = _pcnt(key_hi >= jnp.int16(0), key_hi >= jnp.int16(0))
    ha = jnp.where(ca0 >= need_a, 0, -32768)
    hb = jnp.where(cb0 >= need_b, 0, -32768)

    def hbody(t, carry):
        ha, hb = carry
        step = jnp.int32(1) << (14 - t)
        ta = ha + step
        tb = hb + step
        ca, cb = _pcnt(key_hi >= ta.astype(jnp.int16),
                       key_hi >= tb.astype(jnp.int16))
        ha = jnp.where(ca >= need_a, ta, ha)
        hb = jnp.where(cb >= need_b, tb, hb)
        return ha, hb

    ha, hb = jax.lax.fori_loop(0, 15, hbody, (ha, hb))
    hi_a = ha.astype(jnp.int16)
    hi_b = hb.astype(jnp.int16)

    c_hi_gt_a, c_hi_gt_b = _pcnt(key_hi > hi_a, key_hi > hi_b)
    match_a = key_hi == hi_a
    match_b = key_hi == hi_b
    lo_a = jnp.where(match_a, key_lo, jnp.int16(-32768))
    lo_b = jnp.where(match_b, key_lo, jnp.int16(-32768))
    need_a2 = need_a - c_hi_gt_a
    need_b2 = need_b - c_hi_gt_b

    ca0, cb0 = _pcnt(lo_a >= jnp.int16(0), lo_b >= jnp.int16(0))
    la = jnp.where(ca0 >= need_a2, 0, -32768)
    lb = jnp.where(cb0 >= need_b2, 0, -32768)

    def lbody(t, carry):
        la, lb = carry
        step = jnp.int32(1) << (14 - t)
        ta = la + step
        tb = lb + step
        ca, cb = _pcnt(lo_a >= ta.astype(jnp.int16),
                       lo_b >= tb.astype(jnp.int16))
        la = jnp.where(ca >= need_a2, ta, la)
        lb = jnp.where(cb >= need_b2, tb, lb)
        return la, lb

    la, lb = jax.lax.fori_loop(0, 7, lbody, (la, lb))

    base_a = jax.lax.shift_left(ha, 16) + (la + 32768)
    base_b = jax.lax.shift_left(hb, 16) + (lb + 32768)

    in_a_hi = match_a & (key_lo > (la + (_BRK - 1)).astype(jnp.int16))
    in_a_ge = match_a & (key_lo >= la.astype(jnp.int16))
    in_b_hi = match_b & (key_lo > (lb + (_BRK - 1)).astype(jnp.int16))
    in_b_ge = match_b & (key_lo >= lb.astype(jnp.int16))
    ca1, ca2 = _pcnt(in_a_hi, in_a_ge)
    cb1, cb2 = _pcnt(in_b_hi, in_b_ge)
    c_a_out = c_hi_gt_a + ca1
    t_a_cnt = c_hi_gt_a + ca2
    c_b_out = c_hi_gt_b + cb1
    t_b_cnt = c_hi_gt_b + cb2

    ge_a = key >= base_a
    ge_a_hi = key >= base_a + _BRK
    ge_b = key >= base_b
    ge_b_hi = key >= base_b + _BRK

    m_a = (t_a_cnt - c_a_out).astype(jnp.float32)
    m_b = (t_b_cnt - c_b_out).astype(jnp.float32)
    n_a = (jnp.minimum(t_a_cnt, _K_TOP)
           - jnp.maximum(c_a_out, _K_BOT)).astype(jnp.float32)
    n_b = (jnp.minimum(t_b_cnt, _K_TOP)
           - jnp.maximum(c_b_out, _K_BOT)).astype(jnp.float32)
    same = base_a == base_b

    w_a = n_a / m_a
    w_b = jnp.where(same, 0.0, n_b / m_b)

    a_top = base_a + (_BRK - 1)
    a_f = jax.lax.bitcast_convert_type(
        jnp.where(a_top < 0, a_top ^ jnp.int32(0x7FFFFFFF), a_top),
        jnp.float32)
    m = jnp.maximum(jnp.maximum(l_pos, a_f), _FILL)

    w = jnp.where(ge_a, jnp.where(ge_a_hi, 0.0, w_a),
                  jnp.where(ge_b_hi, 1.0,
                            jnp.where(ge_b, w_b, 0.0)))
    e_term = jnp.where(ge_a_hi, 0.0, w * jnp.exp((sim - m) / _T))
    e_kept = jnp.sum(e_term, axis=1, keepdims=True)

    total = (jnp.exp((l_pos - m) / _T)
             + e_kept
             + jnp.float32(_N - _N_KEPT) * jnp.exp((_FILL - m) / _T))
    out_ref[...] = (m - l_pos) / _T + jnp.log(total)


def kernel(feat_q, feat_k):
    out = pl.pallas_call(
        _loss_kernel,
        grid=(_N // _R,),
        in_specs=[
            pl.BlockSpec((_R, _D), lambda i: (i, 0)),
            pl.BlockSpec((_N, _D), lambda i: (0, 0)),
        ],
        out_specs=pl.BlockSpec((_R, 1), lambda i: (i, 0)),
        out_shape=jax.ShapeDtypeStruct((_N, 1), jnp.float32),
    )(feat_q, feat_k)
    return out.reshape(_N)


# speedup vs baseline: 1.4832x; 1.4832x over previous
import jax
import jax.numpy as jnp
from jax.experimental import pallas as pl

_N = 4096
_D = 64
_R = 512
_T = 0.07
_NUM_NEG = _N - 1
_K_TOP = max(1, int(_NUM_NEG * 0.5))
_K_BOT = max(0, int(_NUM_NEG * 0.1))
_N_KEPT = _K_TOP - _K_BOT
_FILL = -10.0
_INT_MIN = -2147483648


def _pcnt(mask_a, mask_b):
    c = (jnp.where(mask_a, jnp.int16(1), jnp.int16(0))
         + jnp.where(mask_b, jnp.int16(256), jnp.int16(0)))
    n = c.shape[1]
    for _ in range(5):
        n //= 2
        c = c[:, :n] + c[:, n:]
    c32 = c.astype(jnp.int32)
    cb = jnp.sum(jax.lax.shift_right_logical(c32, 8), axis=1, keepdims=True)
    total = jnp.sum(c32, axis=1, keepdims=True)
    ca = total - jax.lax.shift_left(cb, 8)
    return ca, cb


def _loss_kernel(q_ref, k_ref, out_ref):
    i = pl.program_id(0)
    q = q_ref[...]
    k = k_ref[...]
    sim = jax.lax.dot_general(
        q, k, (((1,), (1,)), ((), ())),
        preferred_element_type=jnp.float32,
        precision=jax.lax.Precision.HIGHEST)
    R, N = sim.shape
    row = jax.lax.broadcasted_iota(jnp.int32, (R, N), 0)
    col = jax.lax.broadcasted_iota(jnp.int32, (R, N), 1)
    diag = col == row + i * R
    l_pos = jnp.sum(jnp.where(diag, sim, 0.0), axis=1, keepdims=True)

    bits = jax.lax.bitcast_convert_type(sim, jnp.int32)
    key = jnp.where(bits < 0, bits ^ jnp.int32(0x7FFFFFFF), bits)
    key = jnp.where(diag, jnp.int32(_INT_MIN), key)

    key_hi = jax.lax.shift_right_arithmetic(key, 16).astype(jnp.int16)

    need_a = _K_BOT + 1
    need_b = _K_TOP

    ca0, cb0 = _pcnt(key_hi >= jnp.int16(0), key_hi >= jnp.int16(0))
    ha = jnp.where(ca0 >= need_a, 0, -32768)
    hb = jnp.where(cb0 >= need_b, 0, -32768)

    def hbody(t, carry):
        ha, hb = carry
        step = jnp.int32(1) << (14 - t)
        ta = ha + step
        tb = hb + step
        ca, cb = _pcnt(key_hi >= ta.astype(jnp.int16),
                       key_hi >= tb.astype(jnp.int16))
        ha = jnp.where(ca >= need_a, ta, ha)
        hb = jnp.where(cb >= need_b, tb, hb)
        return ha, hb

    ha, hb = jax.lax.fori_loop(0, 15, hbody, (ha, hb))
    hi_a = ha.astype(jnp.int16)
    hi_b = hb.astype(jnp.int16)

    c_a_out, c_b_out = _pcnt(key_hi > hi_a, key_hi > hi_b)
    ma_c, mb_c = _pcnt(key_hi == hi_a, key_hi == hi_b)
    t_a_cnt = c_a_out + ma_c
    t_b_cnt = c_b_out + mb_c

    m_a = ma_c.astype(jnp.float32)
    m_b = mb_c.astype(jnp.float32)
    n_a = (jnp.minimum(t_a_cnt, _K_TOP)
           - jnp.maximum(c_a_out, _K_BOT)).astype(jnp.float32)
    n_b = (jnp.minimum(t_b_cnt, _K_TOP)
           - jnp.maximum(c_b_out, _K_BOT)).astype(jnp.float32)
    same = ha == hb

    w_a = n_a / m_a
    w_b = jnp.where(same, 0.0, n_b / m_b)

    a_top = jax.lax.shift_left(ha, 16) + 65535
    a_f = jax.lax.bitcast_convert_type(
        jnp.where(a_top < 0, a_top ^ jnp.int32(0x7FFFFFFF), a_top),
        jnp.float32)
    m = jnp.maximum(jnp.maximum(l_pos, a_f), _FILL)

    key_hi32 = jax.lax.shift_right_arithmetic(key, 16)
    ge_a = key_hi32 >= ha
    ge_a_hi = key_hi32 > ha
    ge_b = key_hi32 >= hb
    ge_b_hi = key_hi32 > hb

    w = jnp.where(ge_a, jnp.where(ge_a_hi, 0.0, w_a),
                  jnp.where(ge_b_hi, 1.0,
                            jnp.where(ge_b, w_b, 0.0)))
    e_term = jnp.where(ge_a_hi, 0.0, w * jnp.exp((sim - m) / _T))
    e_kept = jnp.sum(e_term, axis=1, keepdims=True)

    total = (jnp.exp((l_pos - m) / _T)
             + e_kept
             + jnp.float32(_N - _N_KEPT) * jnp.exp((_FILL - m) / _T))
    out_ref[...] = (m - l_pos) / _T + jnp.log(total)


def kernel(feat_q, feat_k):
    out = pl.pallas_call(
        _loss_kernel,
        grid=(_N // _R,),
        in_specs=[
            pl.BlockSpec((_R, _D), lambda i: (i, 0)),
            pl.BlockSpec((_N, _D), lambda i: (0, 0)),
        ],
        out_specs=pl.BlockSpec((_R, 1), lambda i: (i, 0)),
        out_shape=jax.ShapeDtypeStruct((_N, 1), jnp.float32),
    )(feat_q, feat_k)
    return out.reshape(_N)
